# baseline (speedup 1.0000x reference)
import jax
import jax.numpy as jnp
from jax.experimental import pallas as pl
from jax.experimental.pallas import tpu as pltpu


def _dott(a, b, precision=None):
    return jax.lax.dot_general(a, b, (((1,), (1,)), ((), ())),
                               preferred_element_type=jnp.float32,
                               precision=precision)


def _fused_kernel(x0_ref, wr1_ref, bl1_ref, wr2_ref, bl2_ref,
                  wih0_ref, whh0_ref, bi0_ref, bh0_ref,
                  wih1_ref, whh1_ref, bi1_ref, bh1_ref,
                  wfc_ref, bfc_ref,
                  out_ref, gi_ref):
    b, w, f = x0_ref.shape
    h = wr1_ref.shape[1]

    x0 = jnp.transpose(x0_ref[...], (1, 0, 2)).reshape(w * b, f)
    s1 = _dott(x0, wr1_ref[...]) + bl1_ref[...]
    seq = _dott(s1, wr2_ref[...]) + bl2_ref[...]

    gi_ref[...] = (_dott(seq, wih0_ref[...]) + bi0_ref[...]).reshape(w, b, 3 * h)

    whh1 = whh1_ref[...]
    bh0 = bh0_ref[...]
    bh1 = bh1_ref[...]
    bi1 = bi1_ref[...]
    w01 = jnp.concatenate([whh0_ref[...], wih1_ref[...]], axis=0)
    b01 = jnp.concatenate([bh0, bi1], axis=1)

    def gates(gi, gh, hprev):
        r = jax.nn.sigmoid(gi[:, :h] + gh[:, :h])
        z = jax.nn.sigmoid(gi[:, h:2 * h] + gh[:, h:2 * h])
        n = jnp.tanh(gi[:, 2 * h:] + r * gh[:, 2 * h:])
        return (1.0 - z) * n + z * hprev

    zeros = jnp.zeros((b, h), dtype=jnp.float32)

    y_prev = gates(gi_ref[pl.ds(0, 1)].reshape(b, 3 * h), bh0, zeros)

    def one_step(t, h0, h1):
        g01 = _dott(h0, w01) + b01
        gi0 = gi_ref[pl.ds(t, 1)].reshape(b, 3 * h)
        gh1 = _dott(h1, whh1) + bh1
        y_new = gates(gi0, g01[:, :3 * h], h0)
        h1n = gates(g01[:, 3 * h:], gh1, h1)
        return y_new, h1n

    unroll = 9
    def step(i, carry):
        h0, h1 = carry
        for k in range(unroll):
            h0, h1 = one_step(i * unroll + 1 + k, h0, h1)
        return h0, h1

    done = ((w - 1) // unroll) * unroll
    y_prev, h1 = jax.lax.fori_loop(0, (w - 1) // unroll, step, (y_prev, zeros))
    for k in range(w - 1 - done):
        y_prev, h1 = one_step(done + 1 + k, y_prev, h1)

    gi1 = _dott(y_prev, wih1_ref[...]) + bi1
    gh1 = _dott(h1, whh1) + bh1
    h1 = gates(gi1, gh1, h1)

    out_ref[...] = _dott(h1, wfc_ref[...]) + bfc_ref[...]


def kernel(features, Wl1, bl1, Wr1, Wl2, bl2, Wr2, Wih0, Whh0, bih0, bhh0,
           Wih1, Whh1, bih1, bhh1, Wfc, bfc):
    b, w, s, f = features.shape
    h = Wr1.shape[0]
    horizon = Wfc.shape[0]

    args = (features.reshape(b, w, s * f),
            Wr1, bl1.reshape(1, h), Wr2, bl2.reshape(1, h),
            Wih0, Whh0, bih0.reshape(1, 3 * h), bhh0.reshape(1, 3 * h),
            Wih1, Whh1, bih1.reshape(1, 3 * h), bhh1.reshape(1, 3 * h),
            Wfc, bfc.reshape(1, horizon))

    in_specs = [pl.BlockSpec((b, w, f), lambda i: (0, 0, 0))]
    in_specs += [pl.BlockSpec(a.shape, (lambda nd: (lambda i: (0,) * nd))(a.ndim))
                 for a in args[1:]]

    return pl.pallas_call(
        _fused_kernel,
        grid=(1,),
        in_specs=in_specs,
        out_specs=pl.BlockSpec((b, horizon), lambda i: (0, 0)),
        out_shape=jax.ShapeDtypeStruct((b, horizon), jnp.float32),
        scratch_shapes=[pltpu.VMEM((w, b, 3 * h), jnp.float32)],
    )(*args)


# speedup vs baseline: 3.4256x; 3.4256x over previous
import jax
import jax.numpy as jnp
from jax.experimental import pallas as pl
from jax.experimental.pallas import tpu as pltpu


def _dott(a, b, precision=None):
    return jax.lax.dot_general(a, b, (((1,), (1,)), ((), ())),
                               preferred_element_type=jnp.float32,
                               precision=precision)


def _fused_kernel(x0_ref, wr1_ref, bl1_ref, wr2_ref, bl2_ref,
                  wih0_ref, whh0_ref, bi0_ref, bh0_ref,
                  wih1_ref, whh1_ref, bi1_ref, bh1_ref,
                  wfc_ref, bfc_ref,
                  out_ref, gi_ref):
    b, w, _, f = x0_ref.shape
    h = wr1_ref.shape[1]

    x0 = jnp.transpose(x0_ref[:, :, 0, :], (1, 0, 2)).reshape(w * b, f)
    s1 = _dott(x0, wr1_ref[...]) + bl1_ref[...]
    seq = _dott(s1, wr2_ref[...]) + bl2_ref[...]

    gi_ref[...] = (_dott(seq, wih0_ref[...]) + bi0_ref[...]).reshape(w, b, 3 * h)

    whh1 = whh1_ref[...]
    bh0 = bh0_ref[...]
    bh1 = bh1_ref[...]
    bi1 = bi1_ref[...]
    w01 = jnp.concatenate([whh0_ref[...], wih1_ref[...]], axis=0)
    b01 = jnp.concatenate([bh0, bi1], axis=1)

    def gates(gi, gh, hprev):
        r = jax.nn.sigmoid(gi[:, :h] + gh[:, :h])
        z = jax.nn.sigmoid(gi[:, h:2 * h] + gh[:, h:2 * h])
        n = jnp.tanh(gi[:, 2 * h:] + r * gh[:, 2 * h:])
        return (1.0 - z) * n + z * hprev

    zeros = jnp.zeros((b, h), dtype=jnp.float32)

    y_prev = gates(gi_ref[pl.ds(0, 1)].reshape(b, 3 * h), bh0, zeros)

    def one_step(t, h0, h1):
        g01 = _dott(h0, w01) + b01
        gi0 = gi_ref[pl.ds(t, 1)].reshape(b, 3 * h)
        gh1 = _dott(h1, whh1) + bh1
        y_new = gates(gi0, g01[:, :3 * h], h0)
        h1n = gates(g01[:, 3 * h:], gh1, h1)
        return y_new, h1n

    unroll = 9
    def step(i, carry):
        h0, h1 = carry
        for k in range(unroll):
            h0, h1 = one_step(i * unroll + 1 + k, h0, h1)
        return h0, h1

    done = ((w - 1) // unroll) * unroll
    y_prev, h1 = jax.lax.fori_loop(0, (w - 1) // unroll, step, (y_prev, zeros))
    for k in range(w - 1 - done):
        y_prev, h1 = one_step(done + 1 + k, y_prev, h1)

    gi1 = _dott(y_prev, wih1_ref[...]) + bi1
    gh1 = _dott(h1, whh1) + bh1
    h1 = gates(gi1, gh1, h1)

    out_ref[...] = _dott(h1, wfc_ref[...]) + bfc_ref[...]


def kernel(features, Wl1, bl1, Wr1, Wl2, bl2, Wr2, Wih0, Whh0, bih0, bhh0,
           Wih1, Whh1, bih1, bhh1, Wfc, bfc):
    b, w, s, f = features.shape
    h = Wr1.shape[0]
    horizon = Wfc.shape[0]

    args = (features,
            Wr1, bl1.reshape(1, h), Wr2, bl2.reshape(1, h),
            Wih0, Whh0, bih0.reshape(1, 3 * h), bhh0.reshape(1, 3 * h),
            Wih1, Whh1, bih1.reshape(1, 3 * h), bhh1.reshape(1, 3 * h),
            Wfc, bfc.reshape(1, horizon))

    in_specs = [pl.BlockSpec((b, w, 8, f), lambda i: (0, 0, 0, 0))]
    in_specs += [pl.BlockSpec(a.shape, (lambda nd: (lambda i: (0,) * nd))(a.ndim))
                 for a in args[1:]]

    return pl.pallas_call(
        _fused_kernel,
        grid=(1,),
        in_specs=in_specs,
        out_specs=pl.BlockSpec((b, horizon), lambda i: (0, 0)),
        out_shape=jax.ShapeDtypeStruct((b, horizon), jnp.float32),
        scratch_shapes=[pltpu.VMEM((w, b, 3 * h), jnp.float32)],
    )(*args)
